# Initial kernel scaffold; baseline (speedup 1.0000x reference)
import jax
import jax.numpy as jnp
from jax.experimental import pallas as pl


def kernel(x, edge_index, edge_weight, W1, b1, W2, b2):
    raise NotImplementedError("write your pallas kernel here")


# speedup vs baseline: 14.4958x; 14.4958x over previous
import functools

import jax
import jax.numpy as jnp
from jax import lax
from jax.experimental import pallas as pl
from jax.experimental.pallas import tpu as pltpu
from jax.experimental.pallas import tpu_sc as plsc

_NC = 2
_NS = 16
_LANES = 16
_BLK = 128
_SUB = 8
_CHUNK = _BLK * _SUB


def _mm_body(x_ref, w_ref, o_ref):
    o_ref[...] = jnp.dot(x_ref[...], w_ref[...],
                         preferred_element_type=jnp.float32)


def _matmul(x, w, bm):
    n, k = x.shape
    f = w.shape[1]
    return pl.pallas_call(
        _mm_body,
        grid=(pl.cdiv(n, bm),),
        in_specs=[pl.BlockSpec((bm, k), lambda i: (i, 0)),
                  pl.BlockSpec((k, f), lambda i: (0, 0))],
        out_specs=pl.BlockSpec((bm, f), lambda i: (i, 0)),
        out_shape=jax.ShapeDtypeStruct((n, f), jnp.float32),
    )(x, w)


def _make_spmm(n_nodes, e_pad, feat):
    nw = _NC * _NS
    epw = e_pad // nw
    n_chunks = epw // _CHUNK
    rows_per_tile = n_nodes // _NS
    mesh = plsc.VectorSubcoreMesh(core_axis_name="c", subcore_axis_name="s")

    @functools.partial(
        pl.kernel,
        out_type=jax.ShapeDtypeStruct((_NC, n_nodes, feat), jnp.float32),
        mesh=mesh,
        compiler_params=pltpu.CompilerParams(
            needs_layout_passes=False, use_tc_tiling_on_sc=False),
        scratch_types=[
            pltpu.VMEM((_SUB, _BLK), jnp.int32),
            pltpu.VMEM((_SUB, _BLK), jnp.int32),
            pltpu.VMEM((_CHUNK,), jnp.float32),
            pltpu.VMEM((_CHUNK, feat), jnp.float32),
            pltpu.VMEM_SHARED((n_nodes, feat), jnp.float32),
            pltpu.SemaphoreType.DMA,
            pltpu.SemaphoreType.DMA,
        ],
    )
    def spmm(src_hbm, dst_hbm, w_hbm, h_hbm, zeros_hbm, out_hbm,
             src_v, dst_v, w_v, rows_v, acc, sem_e, sem_g):
        c = lax.axis_index("c")
        s = lax.axis_index("s")
        wid = c * _NS + s
        pltpu.sync_copy(zeros_hbm,
                        acc.at[pl.ds(s * rows_per_tile, rows_per_tile)])
        plsc.subcore_barrier()

        base_row = wid * (epw // _BLK)

        def chunk_body(ci, carry):
            row0 = base_row + ci * _SUB
            ce1 = pltpu.async_copy(src_hbm.at[pl.ds(row0, _SUB), :],
                                   src_v, sem_e)
            ce2 = pltpu.async_copy(dst_hbm.at[pl.ds(row0, _SUB), :],
                                   dst_v, sem_e)
            ce3 = pltpu.async_copy(w_hbm.at[pl.ds(row0 * _BLK, _CHUNK)],
                                   w_v, sem_e)
            ce1.wait()
            ce2.wait()
            ce3.wait()
            descs = []
            for j in range(_SUB):
                descs.append(pltpu.async_copy(
                    h_hbm.at[src_v.at[j]],
                    rows_v.at[pl.ds(j * _BLK, _BLK)], sem_g))
            for d in descs:
                d.wait()
            lane = lax.iota(jnp.int32, _LANES)

            def scale_body(e, cy):
                idx = jnp.full((_LANES,), e, jnp.int32)
                wspl = plsc.load_gather(w_v, [idx])
                row = plsc.load_gather(rows_v, [idx, lane])
                plsc.store_scatter(rows_v, [idx, lane], row * wspl)
                return cy

            lax.fori_loop(0, _CHUNK, scale_body, 0, unroll=4)
            for j in range(_SUB):
                pltpu.sync_copy(rows_v.at[pl.ds(j * _BLK, _BLK)],
                                acc.at[dst_v.at[j]], add=True)
            return carry

        lax.fori_loop(0, n_chunks, chunk_body, 0)

        plsc.subcore_barrier()
        pltpu.sync_copy(acc.at[pl.ds(s * rows_per_tile, rows_per_tile)],
                        out_hbm.at[c, pl.ds(s * rows_per_tile, rows_per_tile)])

    return spmm


def _l1_body(p_ref, b_ref, w_ref, o_ref):
    h = jnp.maximum(p_ref[0] + p_ref[1] + b_ref[...], 0.0)
    o_ref[...] = jnp.dot(h, w_ref[...], preferred_element_type=jnp.float32)


def _l2_body(q_ref, b_ref, o_ref):
    z = q_ref[0] + q_ref[1] + b_ref[...]
    col = lax.broadcasted_iota(jnp.int32, z.shape, 1)
    valid = col < 7
    zm = jnp.where(valid, z, -jnp.inf)
    m = jnp.max(zm, axis=1, keepdims=True)
    ez = jnp.where(valid, jnp.exp(z - m), 0.0)
    lse = jnp.log(jnp.sum(ez, axis=1, keepdims=True))
    res = z - m - lse
    o_ref[...] = res[:, :7]


def kernel(x, edge_index, edge_weight, W1, b1, W2, b2):
    n, f_in = x.shape
    f_hid = W1.shape[1]
    f_out = W2.shape[1]
    e = edge_weight.shape[0]
    feat = _LANES
    n_pad = ((n + 127) // 128) * 128

    unit = _NC * _NS * _CHUNK
    e_pad = ((e + unit - 1) // unit) * unit
    pad = e_pad - e
    dst = jnp.pad(edge_index[0], (0, pad)).reshape(e_pad // _BLK, _BLK)
    src = jnp.pad(edge_index[1], (0, pad)).reshape(e_pad // _BLK, _BLK)
    w = jnp.pad(edge_weight, (0, pad))
    zeros = jnp.zeros((n_pad // _NS, feat), jnp.float32)
    w2p = jnp.pad(W2, ((0, 0), (0, feat - f_out)))
    b1r = b1.reshape(1, f_hid)
    b2r = jnp.pad(b2, (0, feat - f_out)).reshape(1, feat)

    spmm = _make_spmm(n_pad, e_pad, feat)

    support1 = _matmul(x, W1, 1024)
    p = spmm(src, dst, w, support1, zeros)

    bm = 2048
    s2 = pl.pallas_call(
        _l1_body,
        grid=(pl.cdiv(n, bm),),
        in_specs=[pl.BlockSpec((_NC, bm, f_hid), lambda i: (0, i, 0)),
                  pl.BlockSpec((1, f_hid), lambda i: (0, 0)),
                  pl.BlockSpec((f_hid, feat), lambda i: (0, 0))],
        out_specs=pl.BlockSpec((bm, feat), lambda i: (i, 0)),
        out_shape=jax.ShapeDtypeStruct((n, feat), jnp.float32),
    )(p, b1r, w2p)

    q = spmm(src, dst, w, s2, zeros)

    out = pl.pallas_call(
        _l2_body,
        grid=(pl.cdiv(n, bm),),
        in_specs=[pl.BlockSpec((_NC, bm, feat), lambda i: (0, i, 0)),
                  pl.BlockSpec((1, feat), lambda i: (0, 0))],
        out_specs=pl.BlockSpec((bm, f_out), lambda i: (i, 0)),
        out_shape=jax.ShapeDtypeStruct((n, f_out), jnp.float32),
    )(q, b2r)
    return out
